# Initial kernel scaffold; baseline (speedup 1.0000x reference)
import jax
import jax.numpy as jnp
from jax.experimental import pallas as pl


def kernel(x, edge_index, Wz, bz, Wr, br, Wh, bh, lzW, lzb, lrW, lrb, lhW, lhb, att, linW, linb):
    raise NotImplementedError("write your pallas kernel here")


# speedup vs baseline: 984.7555x; 984.7555x over previous
import functools

import jax
import jax.numpy as jnp
from jax import lax
from jax.experimental import pallas as pl
from jax.experimental.pallas import tpu as pltpu
from jax.experimental.pallas import tpu_sc as plsc

B = 8
N = 10000
F_IN = 2
F_OUT = 32
PERIODS = 12
HORIZON = 12
E = 160000

C = F_IN * PERIODS * B
HC = C // 2
NP = 10240
EP = 163840
NCORES = 2
NSUB = 16
NW = NCORES * NSUB
EW = EP // NW
K = 128
NCHUNK = EW // K
RPT = NP // NSUB


def _deg_body(dst_hbm, out_hbm, dstv, ones, acc):
    c = lax.axis_index("c")
    s = lax.axis_index("s")
    wid = c * NSUB + s

    one16 = jnp.full((16,), 1.0, jnp.float32)

    def fill(i, carry):
        ones[i, :] = one16
        return carry

    lax.fori_loop(0, K, fill, 0)
    zero16 = jnp.zeros((16,), jnp.float32)

    def zfill(i, carry):
        ones[i, :] = zero16
        return carry

    lax.fori_loop(0, K, zfill, 0)
    for r in range(RPT // K):
        pltpu.sync_copy(ones, acc.at[pl.ds(s * RPT + r * K, K)])
    lax.fori_loop(0, K, fill, 0)
    plsc.subcore_barrier()

    def step(j, carry):
        base = wid * EW + j * K
        pltpu.sync_copy(dst_hbm.at[pl.ds(base, K)], dstv)
        pltpu.sync_copy(ones, acc.at[dstv], add=True)
        return carry

    lax.fori_loop(0, NCHUNK, step, 0)
    plsc.subcore_barrier()
    pltpu.sync_copy(acc.at[pl.ds(s * RPT, RPT)], out_hbm.at[c, pl.ds(s * RPT, RPT)])


def _deg_call(dst):
    fn = pl.kernel(
        _deg_body,
        out_type=jax.ShapeDtypeStruct((NCORES, NP, 16), jnp.float32),
        mesh=plsc.VectorSubcoreMesh(core_axis_name="c", subcore_axis_name="s"),
        scratch_types=[
            pltpu.VMEM((K,), jnp.int32),
            pltpu.VMEM((K, 16), jnp.float32),
            pltpu.VMEM_SHARED((NP, 16), jnp.float32),
        ],
        compiler_params=pltpu.CompilerParams(use_tc_tiling_on_sc=False),
    )
    return fn(dst)


def _make_spmv_body(W):
    def _spmv_body(xs_hbm, src_hbm, dst_hbm, out_hbm, srcv, dstv, buf, acc,
                   sem):
        c = lax.axis_index("c")
        s = lax.axis_index("s")
        wid = c * NSUB + s

        zero16 = jnp.zeros((16,), jnp.float32)

        def zfill(i, carry):
            for k in range(W // 16):
                buf[i, k * 16:(k + 1) * 16] = zero16
            return carry

        lax.fori_loop(0, K, zfill, 0)
        for r in range(RPT // K):
            pltpu.sync_copy(buf, acc.at[pl.ds(s * RPT + r * K, K)])
        plsc.subcore_barrier()

        def step(j, carry):
            base = wid * EW + j * K
            pltpu.sync_copy(src_hbm.at[pl.ds(base, K)], srcv)
            pltpu.sync_copy(dst_hbm.at[pl.ds(base, K)], dstv)
            pltpu.async_copy(xs_hbm.at[srcv], buf, sem).wait()
            pltpu.sync_copy(buf, acc.at[dstv], add=True)
            return carry

        lax.fori_loop(0, NCHUNK, step, 0)
        plsc.subcore_barrier()
        pltpu.sync_copy(acc.at[pl.ds(s * RPT, RPT)],
                        out_hbm.at[c, pl.ds(s * RPT, RPT)])

    return _spmv_body


def _spmv_call(xs, src, dst, W):
    fn = pl.kernel(
        _make_spmv_body(W),
        out_type=jax.ShapeDtypeStruct((NCORES, NP, W), jnp.float32),
        mesh=plsc.VectorSubcoreMesh(core_axis_name="c", subcore_axis_name="s"),
        scratch_types=[
            pltpu.VMEM((K,), jnp.int32),
            pltpu.VMEM((K,), jnp.int32),
            pltpu.VMEM((K, W), jnp.float32),
            pltpu.VMEM_SHARED((NP, W), jnp.float32),
            pltpu.SemaphoreType.DMA,
        ],
        compiler_params=pltpu.CompilerParams(use_tc_tiling_on_sc=False),
    )
    return fn(xs, src, dst)


_NB_SCALE = 2048


def _scale_body(da_ref, db_ref, x_ref, xs0_ref, xs1_ref):
    deg = da_ref[:, 0:1] + db_ref[:, 0:1] + 1.0
    dis = lax.rsqrt(deg)
    xs = x_ref[:, :] * dis
    xs0_ref[:, :] = xs[:, :HC]
    xs1_ref[:, :] = xs[:, HC:]


def _scale_call(dega, degb, x):
    grid = NP // _NB_SCALE
    return pl.pallas_call(
        _scale_body,
        grid=(grid,),
        in_specs=[
            pl.BlockSpec((_NB_SCALE, 16), lambda i: (i, 0)),
            pl.BlockSpec((_NB_SCALE, 16), lambda i: (i, 0)),
            pl.BlockSpec((_NB_SCALE, C), lambda i: (i, 0)),
        ],
        out_specs=[
            pl.BlockSpec((_NB_SCALE, HC), lambda i: (i, 0)),
            pl.BlockSpec((_NB_SCALE, HC), lambda i: (i, 0)),
        ],
        out_shape=[
            jax.ShapeDtypeStruct((NP, HC), jnp.float32),
            jax.ShapeDtypeStruct((NP, HC), jnp.float32),
        ],
    )(dega, degb, x)


_NB_DENSE = 1024


def _dense_body(pa0_ref, pb0_ref, xs0_ref, pa1_ref, pb1_ref, xs1_ref,
                da_ref, db_ref, m16_ref, b16_ref,
                lin_ref, linb_ref, att_ref, out_ref):
    deg = da_ref[:, 0:1] + db_ref[:, 0:1] + 1.0
    dis = lax.rsqrt(deg)
    ax0 = (pa0_ref[:, :] + pb0_ref[:, :] + xs0_ref[:, :]) * dis
    ax1 = (pa1_ref[:, :] + pb1_ref[:, :] + xs1_ref[:, :]) * dis

    a = [att_ref[0, i] for i in range(PERIODS)]
    m = a[0]
    for i in range(1, PERIODS):
        m = jnp.maximum(m, a[i])
    e = [jnp.exp(v - m) for v in a]
    ssum = e[0]
    for i in range(1, PERIODS):
        ssum = ssum + e[i]
    pr = [v / ssum for v in e]

    m16 = m16_ref[:, :]
    b16 = b16_ref[:, :]
    h = jnp.zeros((ax0.shape[0], 256), jnp.float32)
    for p in range(PERIODS):
        axp = jnp.concatenate(
            [ax0[:, p * 8:(p + 1) * 8], ax1[:, p * 8:(p + 1) * 8]],
            axis=1)
        g = jnp.dot(axp, m16, preferred_element_type=jnp.float32) + b16
        z = 1.0 / (1.0 + jnp.exp(-g[:, :256]))
        t = jnp.tanh(g[:, 256:])
        h = h + pr[p] * ((1.0 - z) * t)
    y = jnp.dot(jnp.maximum(h, 0.0), lin_ref[:, :],
                preferred_element_type=jnp.float32) + linb_ref[:, :]
    out_ref[:, :] = y


def _dense_call(pa0, pb0, xs0, pa1, pb1, xs1, dega, degb,
                m16, b16, biglin, linb96, att2):
    grid = NP // _NB_DENSE
    return pl.pallas_call(
        _dense_body,
        grid=(grid,),
        in_specs=[
            pl.BlockSpec((_NB_DENSE, HC), lambda i: (i, 0)),
            pl.BlockSpec((_NB_DENSE, HC), lambda i: (i, 0)),
            pl.BlockSpec((_NB_DENSE, HC), lambda i: (i, 0)),
            pl.BlockSpec((_NB_DENSE, HC), lambda i: (i, 0)),
            pl.BlockSpec((_NB_DENSE, HC), lambda i: (i, 0)),
            pl.BlockSpec((_NB_DENSE, HC), lambda i: (i, 0)),
            pl.BlockSpec((_NB_DENSE, 16), lambda i: (i, 0)),
            pl.BlockSpec((_NB_DENSE, 16), lambda i: (i, 0)),
            pl.BlockSpec((16, 512), lambda i: (0, 0)),
            pl.BlockSpec((1, 512), lambda i: (0, 0)),
            pl.BlockSpec((256, 96), lambda i: (0, 0)),
            pl.BlockSpec((1, 96), lambda i: (0, 0)),
            pl.BlockSpec((1, PERIODS), lambda i: (0, 0),
                         memory_space=pltpu.SMEM),
        ],
        out_specs=pl.BlockSpec((_NB_DENSE, 96), lambda i: (i, 0)),
        out_shape=jax.ShapeDtypeStruct((NP, 96), jnp.float32),
    )(pa0, pb0, xs0, pa1, pb1, xs1, dega, degb,
      m16, b16, biglin, linb96, att2)


def kernel(x, edge_index, Wz, bz, Wr, br, Wh, bh, lzW, lzb, lrW, lrb,
           lhW, lhb, att, linW, linb):
    X = jnp.transpose(x, (1, 2, 3, 0)).reshape(N, C)
    X = jnp.pad(X, ((0, NP - N), (0, 0)))
    pad_idx = jnp.full((EP - E,), NP - 1, jnp.int32)
    src = jnp.concatenate([edge_index[0], pad_idx])
    dst = jnp.concatenate([edge_index[1], pad_idx])

    eye8 = jnp.eye(8, dtype=jnp.float32)
    Mz = Wz @ lzW[:F_OUT]
    cz = bz @ lzW[:F_OUT] + lzb
    Mh = Wh @ lhW[:F_OUT]
    ch = bh @ lhW[:F_OUT] + lhb
    m16_z = jnp.einsum('fo,bc->fbco', Mz, eye8).reshape(16, 256)
    m16_h = jnp.einsum('fo,bc->fbco', Mh, eye8).reshape(16, 256)
    m16 = jnp.concatenate([m16_z, m16_h], axis=1)
    b16 = jnp.concatenate([jnp.tile(cz, 8), jnp.tile(ch, 8)])[None]
    biglin = jnp.einsum('oh,bc->boch', linW, eye8).reshape(256, 96)
    linb96 = jnp.tile(linb, 8)[None]
    att2 = att[None]

    degs = _deg_call(dst)
    dega, degb = degs[0], degs[1]
    xs0, xs1 = _scale_call(dega, degb, X)
    parts0 = _spmv_call(xs0, src, dst, HC)
    parts1 = _spmv_call(xs1, src, dst, HC)
    y = _dense_call(parts0[0], parts0[1], xs0, parts1[0], parts1[1], xs1,
                    dega, degb, m16, b16, biglin, linb96, att2)
    return jnp.transpose(y[:N].reshape(N, B, HORIZON), (1, 0, 2))
